# baseline (device time: 448821 ns/iter reference)
import jax
import jax.numpy as jnp
from jax import lax
from jax.experimental import pallas as pl
from jax.experimental.pallas import tpu as pltpu

N_DEV = 32


def kernel(x, w_mat):
    m_per, k = x.shape
    _, n_per = w_mat.shape
    m_total = N_DEV * m_per

    x = x.astype(jnp.bfloat16)
    w_mat = w_mat.astype(jnp.bfloat16)

    def body(x_ref, w_ref, out_ref, gather_ref, send_sems, recv_sems):
        my = lax.axis_index("i")
        left = lax.rem(my + N_DEV - 1, N_DEV)
        right = lax.rem(my + 1, N_DEV)

        barrier_sem = pltpu.get_barrier_semaphore()
        for nbr in (left, right):
            pl.semaphore_signal(
                barrier_sem, inc=1,
                device_id=(nbr,), device_id_type=pl.DeviceIdType.MESH,
            )
        pl.semaphore_wait(barrier_sem, 2)

        def strip(slot, chunk):
            origin = lax.rem(my + (N_DEV - slot), N_DEV)
            y = jnp.dot(chunk, w_ref[...], preferred_element_type=jnp.float32)
            out_ref[pl.ds(origin * m_per, m_per), :] = jnp.maximum(y, 0.0)

        strip(0, x_ref[...])

        for h in range(1, N_DEV):
            src = x_ref if h == 1 else gather_ref.at[h - 1]
            rdma = pltpu.make_async_remote_copy(
                src_ref=src,
                dst_ref=gather_ref.at[h],
                send_sem=send_sems.at[h - 1],
                recv_sem=recv_sems.at[h - 1],
                device_id=(right,),
                device_id_type=pl.DeviceIdType.MESH,
            )
            rdma.start()
            rdma.wait()
            strip(h, gather_ref[h])

    return pl.pallas_call(
        body,
        out_shape=jax.ShapeDtypeStruct((m_total, n_per), jnp.float32),
        in_specs=[
            pl.BlockSpec(memory_space=pltpu.VMEM),
            pl.BlockSpec(memory_space=pltpu.VMEM),
        ],
        out_specs=pl.BlockSpec(memory_space=pltpu.VMEM),
        scratch_shapes=[
            pltpu.VMEM((N_DEV, m_per, k), jnp.bfloat16),
            pltpu.SemaphoreType.DMA((N_DEV - 1,)),
            pltpu.SemaphoreType.DMA((N_DEV - 1,)),
        ],
        compiler_params=pltpu.CompilerParams(
            collective_id=0,
            vmem_limit_bytes=100 * 1024 * 1024,
        ),
    )(x, w_mat)


# device time: 374538 ns/iter; 1.1983x vs baseline; 1.1983x over previous
import jax
import jax.numpy as jnp
from jax import lax
from jax.experimental import pallas as pl
from jax.experimental.pallas import tpu as pltpu

N_DEV = 32
CW_HOPS = N_DEV // 2
CCW_HOPS = N_DEV - 1 - CW_HOPS


def kernel(x, w_mat):
    m_per, k = x.shape
    _, n_per = w_mat.shape
    m_total = N_DEV * m_per

    x = x.astype(jnp.bfloat16)
    w_mat = w_mat.astype(jnp.bfloat16)

    def body(x_ref, w_ref, out_ref, gather_ref,
             cw_send_sems, cw_recv_sems, ccw_send_sems, ccw_recv_sems):
        my = lax.axis_index("i")
        left = lax.rem(my + N_DEV - 1, N_DEV)
        right = lax.rem(my + 1, N_DEV)

        barrier_sem = pltpu.get_barrier_semaphore()
        for nbr in (left, right):
            pl.semaphore_signal(
                barrier_sem, inc=1,
                device_id=(nbr,), device_id_type=pl.DeviceIdType.MESH,
            )
        pl.semaphore_wait(barrier_sem, 2)

        def make_cw(h):
            src = x_ref if h == 1 else gather_ref.at[h - 1]
            return pltpu.make_async_remote_copy(
                src_ref=src,
                dst_ref=gather_ref.at[h],
                send_sem=cw_send_sems.at[h - 1],
                recv_sem=cw_recv_sems.at[h - 1],
                device_id=(right,),
                device_id_type=pl.DeviceIdType.MESH,
            )

        def make_ccw(g):
            src = x_ref if g == 1 else gather_ref.at[33 - g]
            return pltpu.make_async_remote_copy(
                src_ref=src,
                dst_ref=gather_ref.at[32 - g],
                send_sem=ccw_send_sems.at[g - 1],
                recv_sem=ccw_recv_sems.at[g - 1],
                device_id=(left,),
                device_id_type=pl.DeviceIdType.MESH,
            )

        def strip(slot, chunk):
            origin = lax.rem(my + (N_DEV - slot), N_DEV)
            y = jnp.dot(chunk, w_ref[...], preferred_element_type=jnp.float32)
            out_ref[pl.ds(origin * m_per, m_per), :] = jnp.maximum(y, 0.0)

        cw = {h: make_cw(h) for h in range(1, CW_HOPS + 1)}
        ccw = {g: make_ccw(g) for g in range(1, CCW_HOPS + 1)}

        cw[1].start()
        ccw[1].start()
        strip(0, x_ref[...])

        for step in range(1, CW_HOPS + 1):
            cw[step].wait_recv()
            if step + 1 <= CW_HOPS:
                cw[step + 1].start()
            if step <= CCW_HOPS:
                ccw[step].wait_recv()
                if step + 1 <= CCW_HOPS:
                    ccw[step + 1].start()
            strip(step, gather_ref[step])
            if step <= CCW_HOPS:
                strip(32 - step, gather_ref[32 - step])

        for h in range(1, CW_HOPS + 1):
            cw[h].wait_send()
        for g in range(1, CCW_HOPS + 1):
            ccw[g].wait_send()

    return pl.pallas_call(
        body,
        out_shape=jax.ShapeDtypeStruct((m_total, n_per), jnp.float32),
        in_specs=[
            pl.BlockSpec(memory_space=pltpu.VMEM),
            pl.BlockSpec(memory_space=pltpu.VMEM),
        ],
        out_specs=pl.BlockSpec(memory_space=pltpu.VMEM),
        scratch_shapes=[
            pltpu.VMEM((N_DEV, m_per, k), jnp.bfloat16),
            pltpu.SemaphoreType.DMA((CW_HOPS,)),
            pltpu.SemaphoreType.DMA((CW_HOPS,)),
            pltpu.SemaphoreType.DMA((CCW_HOPS,)),
            pltpu.SemaphoreType.DMA((CCW_HOPS,)),
        ],
        compiler_params=pltpu.CompilerParams(
            collective_id=0,
            vmem_limit_bytes=100 * 1024 * 1024,
        ),
    )(x, w_mat)


# device time: 228548 ns/iter; 1.9638x vs baseline; 1.6388x over previous
import numpy as np

import jax
import jax.numpy as jnp
from jax import lax
from jax.experimental import pallas as pl
from jax.experimental.pallas import tpu as pltpu

N_DEV = 32
CW_HOPS = 16
CCW_HOPS = 15

_MESH_COORDS = []
for _z in range(4):
    for _yi, _y in enumerate(range(4)):
        _row = [(0, _y, _z), (1, _y, _z)]
        if _yi % 2:
            _row.reverse()
        _MESH_COORDS.extend(_row)
_COORD_TO_MESH = {c: i for i, c in enumerate(_MESH_COORDS)}

_P = []
for _z in range(4):
    _ys = range(4) if _z % 2 == 0 else range(3, -1, -1)
    _P.extend((_y, _z) for _y in _ys)
_CYCLE_COORDS = [(0, y, z) for (y, z) in _P] + [(1, y, z) for (y, z) in reversed(_P)]
_CYCLE_MESH = [_COORD_TO_MESH[c] for c in _CYCLE_COORDS]
_POS = {m: p for p, m in enumerate(_CYCLE_MESH)}

_META = np.zeros((N_DEV, 2 + CW_HOPS + CCW_HOPS), np.int32)
for _d in range(N_DEV):
    _p = _POS[_d]
    _META[_d, 0] = _CYCLE_MESH[(_p - 1) % N_DEV]
    _META[_d, 1] = _CYCLE_MESH[(_p + 1) % N_DEV]
    for _h in range(1, CW_HOPS + 1):
        _META[_d, 1 + _h] = _CYCLE_MESH[(_p - _h) % N_DEV]
    for _g in range(1, CCW_HOPS + 1):
        _META[_d, 1 + CW_HOPS + _g] = _CYCLE_MESH[(_p + _g) % N_DEV]


def kernel(x, w_mat):
    m_per, k = x.shape
    _, n_per = w_mat.shape
    m_total = N_DEV * m_per

    x = x.astype(jnp.bfloat16)
    w_mat = w_mat.astype(jnp.bfloat16)

    my = lax.axis_index("i")
    meta = jnp.asarray(_META)[my]

    def body(x_ref, w_ref, meta_ref, out_ref, gather_ref,
             cw_send_sems, cw_recv_sems, ccw_send_sems, ccw_recv_sems):
        left = meta_ref[0]
        right = meta_ref[1]

        barrier_sem = pltpu.get_barrier_semaphore()
        for nbr in (left, right):
            pl.semaphore_signal(
                barrier_sem, inc=1,
                device_id=(nbr,), device_id_type=pl.DeviceIdType.MESH,
            )
        pl.semaphore_wait(barrier_sem, 2)

        def make_cw(h):
            src = x_ref if h == 1 else gather_ref.at[h - 1]
            return pltpu.make_async_remote_copy(
                src_ref=src,
                dst_ref=gather_ref.at[h],
                send_sem=cw_send_sems.at[h - 1],
                recv_sem=cw_recv_sems.at[h - 1],
                device_id=(right,),
                device_id_type=pl.DeviceIdType.MESH,
            )

        def make_ccw(g):
            src = x_ref if g == 1 else gather_ref.at[33 - g]
            return pltpu.make_async_remote_copy(
                src_ref=src,
                dst_ref=gather_ref.at[32 - g],
                send_sem=ccw_send_sems.at[g - 1],
                recv_sem=ccw_recv_sems.at[g - 1],
                device_id=(left,),
                device_id_type=pl.DeviceIdType.MESH,
            )

        def strip(origin, chunk):
            y = jnp.dot(chunk, w_ref[...], preferred_element_type=jnp.float32)
            out_ref[pl.ds(origin * m_per, m_per), :] = jnp.maximum(y, 0.0)

        cw = {h: make_cw(h) for h in range(1, CW_HOPS + 1)}
        ccw = {g: make_ccw(g) for g in range(1, CCW_HOPS + 1)}

        cw[1].start()
        ccw[1].start()
        strip(lax.axis_index("i"), x_ref[...])

        for step in range(1, CW_HOPS + 1):
            cw[step].wait_recv()
            if step + 1 <= CW_HOPS:
                cw[step + 1].start()
            if step <= CCW_HOPS:
                ccw[step].wait_recv()
                if step + 1 <= CCW_HOPS:
                    ccw[step + 1].start()
            strip(meta_ref[1 + step], gather_ref[step])
            if step <= CCW_HOPS:
                strip(meta_ref[1 + CW_HOPS + step], gather_ref[32 - step])

        for h in range(1, CW_HOPS + 1):
            cw[h].wait_send()
        for g in range(1, CCW_HOPS + 1):
            ccw[g].wait_send()

    return pl.pallas_call(
        body,
        out_shape=jax.ShapeDtypeStruct((m_total, n_per), jnp.float32),
        in_specs=[
            pl.BlockSpec(memory_space=pltpu.VMEM),
            pl.BlockSpec(memory_space=pltpu.VMEM),
            pl.BlockSpec(memory_space=pltpu.SMEM),
        ],
        out_specs=pl.BlockSpec(memory_space=pltpu.VMEM),
        scratch_shapes=[
            pltpu.VMEM((N_DEV, m_per, k), jnp.bfloat16),
            pltpu.SemaphoreType.DMA((CW_HOPS,)),
            pltpu.SemaphoreType.DMA((CW_HOPS,)),
            pltpu.SemaphoreType.DMA((CCW_HOPS,)),
            pltpu.SemaphoreType.DMA((CCW_HOPS,)),
        ],
        compiler_params=pltpu.CompilerParams(
            collective_id=0,
            vmem_limit_bytes=100 * 1024 * 1024,
        ),
    )(x, w_mat, meta)


# device time: 203505 ns/iter; 2.2055x vs baseline; 1.1231x over previous
import numpy as np

import jax
import jax.numpy as jnp
from jax import lax
from jax.experimental import pallas as pl
from jax.experimental.pallas import tpu as pltpu

N_DEV = 32
CW_HOPS = 16
CCW_HOPS = 15

_MESH_COORDS = []
for _z in range(4):
    for _yi, _y in enumerate(range(4)):
        _row = [(0, _y, _z), (1, _y, _z)]
        if _yi % 2:
            _row.reverse()
        _MESH_COORDS.extend(_row)
_COORD_TO_MESH = {c: i for i, c in enumerate(_MESH_COORDS)}

_P = []
for _z in range(4):
    _ys = range(4) if _z % 2 == 0 else range(3, -1, -1)
    _P.extend((_y, _z) for _y in _ys)
_CYCLE_COORDS = [(0, y, z) for (y, z) in _P] + [(1, y, z) for (y, z) in reversed(_P)]
_CYCLE_MESH = [_COORD_TO_MESH[c] for c in _CYCLE_COORDS]
_POS = {m: p for p, m in enumerate(_CYCLE_MESH)}

_META = np.zeros((N_DEV, 2 + CW_HOPS + CCW_HOPS), np.int32)
for _d in range(N_DEV):
    _p = _POS[_d]
    _META[_d, 0] = _CYCLE_MESH[(_p - 1) % N_DEV]
    _META[_d, 1] = _CYCLE_MESH[(_p + 1) % N_DEV]
    for _h in range(1, CW_HOPS + 1):
        _META[_d, 1 + _h] = _CYCLE_MESH[(_p - _h) % N_DEV]
    for _g in range(1, CCW_HOPS + 1):
        _META[_d, 1 + CW_HOPS + _g] = _CYCLE_MESH[(_p + _g) % N_DEV]


def kernel(x, w_mat):
    m_per, k = x.shape
    _, n_per = w_mat.shape
    m_total = N_DEV * m_per

    x = x.astype(jnp.bfloat16)
    w_mat = w_mat.astype(jnp.bfloat16)

    my = lax.axis_index("i")
    meta = jnp.asarray(_META)[my]

    n_sub = 4
    rows_sub = m_per // n_sub

    def body(x_ref, w_ref, meta_ref, out_ref, gather_ref,
             cw_send_sems, cw_recv_sems, ccw_send_sems, ccw_recv_sems):
        left = meta_ref[0]
        right = meta_ref[1]

        barrier_sem = pltpu.get_barrier_semaphore()
        for nbr in (left, right):
            pl.semaphore_signal(
                barrier_sem, inc=1,
                device_id=(nbr,), device_id_type=pl.DeviceIdType.MESH,
            )
        pl.semaphore_wait(barrier_sem, 2)

        def sub(ref, j):
            return ref.at[pl.ds(j * rows_sub, rows_sub)]

        def make_cw(h, j):
            src = x_ref if h == 1 else gather_ref.at[h - 1]
            return pltpu.make_async_remote_copy(
                src_ref=sub(src, j),
                dst_ref=sub(gather_ref.at[h], j),
                send_sem=cw_send_sems.at[h - 1, j],
                recv_sem=cw_recv_sems.at[h - 1, j],
                device_id=(right,),
                device_id_type=pl.DeviceIdType.MESH,
            )

        def make_ccw(g, j):
            src = x_ref if g == 1 else gather_ref.at[33 - g]
            return pltpu.make_async_remote_copy(
                src_ref=sub(src, j),
                dst_ref=sub(gather_ref.at[32 - g], j),
                send_sem=ccw_send_sems.at[g - 1, j],
                recv_sem=ccw_recv_sems.at[g - 1, j],
                device_id=(left,),
                device_id_type=pl.DeviceIdType.MESH,
            )

        def strip(origin, chunk):
            y = jnp.dot(chunk, w_ref[...], preferred_element_type=jnp.float32)
            out_ref[pl.ds(origin * m_per, m_per), :] = jnp.maximum(y, 0.0)

        cw = {(h, j): make_cw(h, j)
              for h in range(1, CW_HOPS + 1) for j in range(n_sub)}
        ccw = {(g, j): make_ccw(g, j)
               for g in range(1, CCW_HOPS + 1) for j in range(n_sub)}

        for j in range(n_sub):
            cw[1, j].start()
            ccw[1, j].start()
        strip(lax.axis_index("i"), x_ref[...])

        for step in range(1, CW_HOPS + 1):
            for j in range(n_sub):
                cw[step, j].wait_recv()
                if step + 1 <= CW_HOPS:
                    cw[step + 1, j].start()
                if step <= CCW_HOPS:
                    ccw[step, j].wait_recv()
                    if step + 1 <= CCW_HOPS:
                        ccw[step + 1, j].start()
            strip(meta_ref[1 + step], gather_ref[step])
            if step <= CCW_HOPS:
                strip(meta_ref[1 + CW_HOPS + step], gather_ref[32 - step])

        for h in range(1, CW_HOPS + 1):
            for j in range(n_sub):
                cw[h, j].wait_send()
        for g in range(1, CCW_HOPS + 1):
            for j in range(n_sub):
                ccw[g, j].wait_send()

    return pl.pallas_call(
        body,
        out_shape=jax.ShapeDtypeStruct((m_total, n_per), jnp.float32),
        in_specs=[
            pl.BlockSpec(memory_space=pltpu.VMEM),
            pl.BlockSpec(memory_space=pltpu.VMEM),
            pl.BlockSpec(memory_space=pltpu.SMEM),
        ],
        out_specs=pl.BlockSpec(memory_space=pltpu.VMEM),
        scratch_shapes=[
            pltpu.VMEM((N_DEV, m_per, k), jnp.bfloat16),
            pltpu.SemaphoreType.DMA((CW_HOPS, 4)),
            pltpu.SemaphoreType.DMA((CW_HOPS, 4)),
            pltpu.SemaphoreType.DMA((CCW_HOPS, 4)),
            pltpu.SemaphoreType.DMA((CCW_HOPS, 4)),
        ],
        compiler_params=pltpu.CompilerParams(
            collective_id=0,
            vmem_limit_bytes=100 * 1024 * 1024,
        ),
    )(x, w_mat, meta)
